# Initial kernel scaffold; baseline (speedup 1.0000x reference)
import jax
import jax.numpy as jnp
from jax.experimental import pallas as pl


def kernel(inputs):
    raise NotImplementedError("write your pallas kernel here")


# speedup vs baseline: 16.9690x; 16.9690x over previous
import jax
import jax.numpy as jnp
from jax.experimental import pallas as pl
from jax.experimental.pallas import tpu as pltpu

_TOPK = 64
_BLOCK_ROWS = 64


def _topk_mask_block(x_ref, o_ref):
    x = x_ref[...]
    u = jax.lax.bitcast_convert_type(x, jnp.int32)
    key = u ^ (jnp.right_shift(u, 31) & jnp.int32(0x7FFFFFFF))
    cnt_pos = jnp.sum((key >= 0).astype(jnp.int32), axis=1, keepdims=True)
    t = jnp.where(cnt_pos >= _TOPK, jnp.int32(0), jnp.int32(-(2 ** 31)))
    for b in range(30, -1, -1):
        cand = t | jnp.int32(1 << b)
        cnt = jnp.sum((key >= cand).astype(jnp.int32), axis=1, keepdims=True)
        t = jnp.where(cnt >= _TOPK, cand, t)
    o_ref[...] = jnp.where(key >= t, x, jnp.float32(0.0))


def kernel(inputs):
    x = inputs
    rows, cols = x.shape
    br = min(_BLOCK_ROWS, rows)
    return pl.pallas_call(
        _topk_mask_block,
        grid=(rows // br,),
        in_specs=[pl.BlockSpec((br, cols), lambda i: (i, 0))],
        out_specs=pl.BlockSpec((br, cols), lambda i: (i, 0)),
        out_shape=jax.ShapeDtypeStruct((rows, cols), jnp.float32),
        compiler_params=pltpu.CompilerParams(
            dimension_semantics=("arbitrary",)),
    )(x)
